# baseline (speedup 1.0000x reference)
import functools

import jax
import jax.numpy as jnp
from jax import lax
from jax.experimental import pallas as pl
from jax.experimental.pallas import tpu as pltpu
from jax.experimental.pallas import tpu_sc as plsc

L = 16

B = 8
N = 4096
NP = 2048
DV = 64
NQ = 4


def _bf16_round(v):
  u = plsc.bitcast(v, jnp.uint32)
  r = u + jnp.uint32(0x7FFF) + ((u >> jnp.uint32(16)) & jnp.uint32(1))
  return plsc.bitcast(r & jnp.uint32(0xFFFF0000), jnp.float32)


def _make_body(b_sz, n, npoint, dv):
  steps = n // L
  nq_pts = n // NQ
  qsteps = nq_pts // L
  opq = npoint // NQ
  slot = 8

  def _fps_body(ct_hbm, vt_hbm, mt_hbm, oc_hbm, ov_hbm, om_hbm,
                xs, ys, zs, xb, yb, zb, sqs, dist, msk, idxb, idx2,
                oxs, oys, ozs, oms, xbuf, rbuf, vbuf, shr, sem):
    c = lax.axis_index("c")
    s = lax.axis_index("s")
    b = c * NQ + s // NQ
    q = s % NQ
    g0 = (s // NQ) * NQ

    pltpu.sync_copy(ct_hbm.at[pl.ds((b * 3 + 0) * n, n)], xs)
    pltpu.sync_copy(ct_hbm.at[pl.ds((b * 3 + 1) * n, n)], ys)
    pltpu.sync_copy(ct_hbm.at[pl.ds((b * 3 + 2) * n, n)], zs)
    pltpu.sync_copy(mt_hbm.at[pl.ds(b * n, n)], msk)

    big = jnp.full((L,), 1e10, jnp.float32)

    def init_step(t, carry):
      sl = pl.ds(t * L, L)
      x = xs[sl]
      y = ys[sl]
      z = zs[sl]
      sqs[sl] = (x * x + y * y) + z * z
      xb[sl] = _bf16_round(x)
      yb[sl] = _bf16_round(y)
      zb[sl] = _bf16_round(z)
      return carry

    lax.fori_loop(0, steps, init_step, 0)

    def init_dist(t, carry):
      dist[pl.ds(t * L, L)] = big
      return carry

    lax.fori_loop(0, qsteps, init_dist, 0)

    iota = lax.iota(jnp.int32, L)
    magm = jnp.uint32(0x7FFFFFFF)
    expm = jnp.uint32(0x7F800000)
    emin = jnp.uint32(28 << 23)
    einv = jnp.uint32(281 << 23)
    eneg2 = jnp.uint32(0x73000000)
    qbase = q * qsteps

    def fps_block(blk, far0):
      def fps_iter(j, carry):
        far, acc = carry
        cvec = jnp.full((L,), far, jnp.int32)
        acc = jnp.where(iota == j, cvec, acc)
        cx = plsc.load_gather(xs, [cvec])
        cy = plsc.load_gather(ys, [cvec])
        cz = plsc.load_gather(zs, [cvec])
        cxb = plsc.load_gather(xb, [cvec])
        cyb = plsc.load_gather(yb, [cvec])
        czb = plsc.load_gather(zb, [cvec])
        csq = (cx * cx + cz * cz) + cy * cy

        def sweep(t, carry2):
          rmax, ridx = carry2
          gsl = pl.ds((qbase + t) * L, L)
          p0 = xb[gsl] * cxb
          p1 = yb[gsl] * cyb
          p2 = zb[gsl] * czb
          u0 = plsc.bitcast(p0, jnp.uint32) & magm
          u1 = plsc.bitcast(p1, jnp.uint32) & magm
          u2 = plsc.bitcast(p2, jnp.uint32) & magm
          e = jnp.maximum(jnp.maximum(u0, u1), u2) & expm
          e = jnp.maximum(e, emin)
          inv = plsc.bitcast(einv - e, jnp.float32)
          q0 = (p0 * inv).astype(jnp.int32)
          q1 = (p1 * inv).astype(jnp.int32)
          q2 = (p2 * inv).astype(jnp.int32)
          md = ((q0 + (q1 + q2)).astype(jnp.float32)
                * plsc.bitcast(e + eneg2, jnp.float32))
          d = (md + sqs[gsl]) + csq
          sl = pl.ds(t * L, L)
          nd = jnp.minimum(dist[sl], d)
          dist[sl] = nd
          m = nd > rmax
          rmax = jnp.where(m, nd, rmax)
          ridx = jnp.where(m, jnp.full((L,), qbase + t, jnp.int32), ridx)
          return (rmax, ridx)

        rmax, ridx = plsc.parallel_loop(
            0, qsteps, 1, unroll=10,
            carry=(jnp.full((L,), -1.0, jnp.float32),
                   jnp.zeros((L,), jnp.int32)))(sweep)
        gidx = ridx * L + iota

        mx = jnp.max(rmax)
        mi = jnp.min(jnp.where(rmax == mx, gidx, jnp.int32(n)))

        parity = j & 1
        mxi = lax.bitcast_convert_type(mx, jnp.int32)
        pair = jnp.where(iota == 0, jnp.full((L,), mxi, jnp.int32),
                         jnp.full((L,), mi, jnp.int32))
        xbuf[pl.ds(0, L)] = pair
        off = parity * (16 * slot)
        pltpu.sync_copy(xbuf.at[pl.ds(0, slot)],
                        shr.at[pl.ds(off + s * slot, slot)])
        plsc.subcore_barrier()
        pltpu.sync_copy(shr.at[pl.ds(off + g0 * slot, NQ * slot)], rbuf)

        v0i = rbuf[pl.ds(0, L)]
        v1i = rbuf[pl.ds(L, L)]
        v0f = plsc.bitcast(v0i, jnp.float32)
        v1f = plsc.bitcast(v1i, jnp.float32)
        vals = (v0f[0], v0f[slot], v1f[0], v1f[slot])
        idxs = (v0i[1], v0i[slot + 1], v1i[1], v1i[slot + 1])
        bv, bi = vals[0], idxs[0]
        for qq in range(1, NQ):
          ov, oi = vals[qq], idxs[qq]
          take = (ov > bv) | ((ov == bv) & (oi < bi))
          bv = jnp.where(take, ov, bv)
          bi = jnp.where(take, oi, bi)
        return (bi, acc)

      far, acc = lax.fori_loop(
          0, L, fps_iter, (far0, jnp.zeros((L,), jnp.int32)))
      idxb[pl.ds(blk * L, L)] = acc
      return far

    lax.fori_loop(0, npoint // L, fps_block, jnp.int32(0))

    obase = q * opq

    def gather_step(j, carry):
      sl = pl.ds(j * L, L)
      iv = idxb[pl.ds(obase + j * L, L)]
      oxs[sl] = plsc.load_gather(xs, [iv])
      oys[sl] = plsc.load_gather(ys, [iv])
      ozs[sl] = plsc.load_gather(zs, [iv])
      oms[sl] = plsc.load_gather(msk, [iv])
      idx2[sl] = iv + b * n
      return carry

    lax.fori_loop(0, opq // L, gather_step, 0)

    pltpu.sync_copy(oxs, oc_hbm.at[pl.ds((b * 3 + 0) * npoint + obase, opq)])
    pltpu.sync_copy(oys, oc_hbm.at[pl.ds((b * 3 + 1) * npoint + obase, opq)])
    pltpu.sync_copy(ozs, oc_hbm.at[pl.ds((b * 3 + 2) * npoint + obase, opq)])
    pltpu.sync_copy(oms, om_hbm.at[pl.ds(b * npoint + obase, opq)])

    pltpu.async_copy(vt_hbm.at[idx2], vbuf, sem).wait()
    pltpu.sync_copy(vbuf, ov_hbm.at[b, pl.ds(obase, opq)])

  return _fps_body


def _make_call(b_sz, n, npoint, dv, interpret=False):
  mesh = plsc.VectorSubcoreMesh(
      core_axis_name="c", subcore_axis_name="s", num_cores=2, num_subcores=16)
  nq_pts = n // NQ
  opq = npoint // NQ
  slot = 8
  return pl.kernel(
      _make_body(b_sz, n, npoint, dv),
      out_type=[
          jax.ShapeDtypeStruct((b_sz * 3 * npoint,), jnp.float32),
          jax.ShapeDtypeStruct((b_sz, npoint, dv), jnp.float32),
          jax.ShapeDtypeStruct((b_sz * npoint,), jnp.float32),
      ],
      mesh=mesh,
      scratch_types=[
          pltpu.VMEM((n,), jnp.float32),
          pltpu.VMEM((n,), jnp.float32),
          pltpu.VMEM((n,), jnp.float32),
          pltpu.VMEM((n,), jnp.float32),
          pltpu.VMEM((n,), jnp.float32),
          pltpu.VMEM((n,), jnp.float32),
          pltpu.VMEM((n,), jnp.float32),
          pltpu.VMEM((nq_pts,), jnp.float32),
          pltpu.VMEM((n,), jnp.float32),
          pltpu.VMEM((npoint,), jnp.int32),
          pltpu.VMEM((opq,), jnp.int32),
          pltpu.VMEM((opq,), jnp.float32),
          pltpu.VMEM((opq,), jnp.float32),
          pltpu.VMEM((opq,), jnp.float32),
          pltpu.VMEM((opq,), jnp.float32),
          pltpu.VMEM((L,), jnp.int32),
          pltpu.VMEM((NQ * slot,), jnp.int32),
          pltpu.VMEM((opq, dv), jnp.float32),
          pltpu.VMEM_SHARED((2 * 16 * slot,), jnp.int32),
          pltpu.SemaphoreType.DMA,
      ],
      compiler_params=pltpu.CompilerParams(
          needs_layout_passes=False, use_tc_tiling_on_sc=False),
      interpret=interpret,
  )


@jax.jit
def _fps_call(ct, vt, mt):
  return _make_call(B, N, NP, DV)(ct, vt, mt)


def kernel(coords, values, mask):
  ct = coords.transpose(0, 2, 1).reshape(B * 3 * N)
  vt = values.reshape(B * N, DV)
  mt = mask.reshape(B * N)
  oc, ov, om = _fps_call(ct, vt, mt)
  return (oc.reshape(B, 3, NP).transpose(0, 2, 1), ov,
          om.reshape(B, NP, 1))


# speedup vs baseline: 1.0672x; 1.0672x over previous
import functools

import jax
import jax.numpy as jnp
from jax import lax
from jax.experimental import pallas as pl
from jax.experimental.pallas import tpu as pltpu
from jax.experimental.pallas import tpu_sc as plsc

L = 16

B = 8
N = 4096
NP = 2048
DV = 64
NQ = 4


def _bf16_round(v):
  u = plsc.bitcast(v, jnp.uint32)
  r = u + jnp.uint32(0x7FFF) + ((u >> jnp.uint32(16)) & jnp.uint32(1))
  return plsc.bitcast(r & jnp.uint32(0xFFFF0000), jnp.float32)


def _make_body(b_sz, n, npoint, dv):
  steps = n // L
  nq_pts = n // NQ
  qsteps = nq_pts // L
  opq = npoint // NQ
  slot = 8

  def _fps_body(ct_hbm, vt_hbm, mt_hbm, oc_hbm, ov_hbm, om_hbm,
                xs, ys, zs, xb, yb, zb, sqs, dist, msk, idxb, idx2,
                oxs, oys, ozs, oms, xbuf, rbuf, vbuf, shr, sem):
    c = lax.axis_index("c")
    s = lax.axis_index("s")
    b = c * NQ + s // NQ
    q = s % NQ
    g0 = (s // NQ) * NQ

    pltpu.sync_copy(ct_hbm.at[pl.ds((b * 3 + 0) * n, n)], xs)
    pltpu.sync_copy(ct_hbm.at[pl.ds((b * 3 + 1) * n, n)], ys)
    pltpu.sync_copy(ct_hbm.at[pl.ds((b * 3 + 2) * n, n)], zs)
    pltpu.sync_copy(mt_hbm.at[pl.ds(b * n, n)], msk)

    big = jnp.full((L,), 1e10, jnp.float32)

    def init_step(t, carry):
      sl = pl.ds(t * L, L)
      x = xs[sl]
      y = ys[sl]
      z = zs[sl]
      sqs[sl] = (x * x + y * y) + z * z
      xb[sl] = _bf16_round(x)
      yb[sl] = _bf16_round(y)
      zb[sl] = _bf16_round(z)
      return carry

    lax.fori_loop(0, steps, init_step, 0)

    def init_dist(t, carry):
      dist[pl.ds(t * L, L)] = big
      return carry

    lax.fori_loop(0, qsteps, init_dist, 0)

    iota = lax.iota(jnp.int32, L)
    magm = jnp.uint32(0x7FFFFFFF)
    expm = jnp.uint32(0x7F800000)
    emin = jnp.uint32(28 << 23)
    einv = jnp.uint32(281 << 23)
    eneg2 = jnp.uint32(0x73000000)
    qbase = q * qsteps

    def fps_block(blk, far0):
      def fps_iter(j, carry):
        far, acc = carry
        cvec = jnp.full((L,), far, jnp.int32)
        acc = jnp.where(iota == j, cvec, acc)
        cx = plsc.load_gather(xs, [cvec])
        cy = plsc.load_gather(ys, [cvec])
        cz = plsc.load_gather(zs, [cvec])
        cxb = plsc.load_gather(xb, [cvec])
        cyb = plsc.load_gather(yb, [cvec])
        czb = plsc.load_gather(zb, [cvec])
        csq = (cx * cx + cz * cz) + cy * cy

        def sweep(t, carry2):
          rmax, ridx = carry2
          gsl = pl.ds((qbase + t) * L, L)
          p0 = xb[gsl] * cxb
          p1 = yb[gsl] * cyb
          p2 = zb[gsl] * czb
          u0 = plsc.bitcast(p0, jnp.uint32) & magm
          u1 = plsc.bitcast(p1, jnp.uint32) & magm
          u2 = plsc.bitcast(p2, jnp.uint32) & magm
          e = jnp.maximum(jnp.maximum(u0, u1), u2) & expm
          e = jnp.maximum(e, emin)
          inv = plsc.bitcast(einv - e, jnp.float32)
          q0 = (p0 * inv).astype(jnp.int32)
          q1 = (p1 * inv).astype(jnp.int32)
          q2 = (p2 * inv).astype(jnp.int32)
          md = ((q0 + (q1 + q2)).astype(jnp.float32)
                * plsc.bitcast(e + eneg2, jnp.float32))
          d = (md + sqs[gsl]) + csq
          sl = pl.ds(t * L, L)
          nd = jnp.minimum(dist[sl], d)
          dist[sl] = nd
          m = nd > rmax
          rmax = jnp.where(m, nd, rmax)
          ridx = jnp.where(m, jnp.full((L,), qbase + t, jnp.int32), ridx)
          return (rmax, ridx)

        rmax, ridx = plsc.parallel_loop(
            0, qsteps, 1, unroll=8,
            carry=(jnp.full((L,), -1.0, jnp.float32),
                   jnp.zeros((L,), jnp.int32)))(sweep)
        gidx = ridx * L + iota

        mx = jnp.max(rmax)
        mi = jnp.min(jnp.where(rmax == mx, gidx, jnp.int32(n)))

        parity = j & 1
        mxi = lax.bitcast_convert_type(mx, jnp.int32)
        pair = jnp.where(iota == 0, jnp.full((L,), mxi, jnp.int32),
                         jnp.full((L,), mi, jnp.int32))
        xbuf[pl.ds(0, L)] = pair
        off = parity * (16 * slot)
        pltpu.sync_copy(xbuf.at[pl.ds(0, slot)],
                        shr.at[pl.ds(off + s * slot, slot)])
        plsc.subcore_barrier()
        pltpu.sync_copy(shr.at[pl.ds(off + g0 * slot, NQ * slot)], rbuf)

        v0i = rbuf[pl.ds(0, L)]
        v1i = rbuf[pl.ds(L, L)]
        v0f = plsc.bitcast(v0i, jnp.float32)
        v1f = plsc.bitcast(v1i, jnp.float32)
        vals = (v0f[0], v0f[slot], v1f[0], v1f[slot])
        idxs = (v0i[1], v0i[slot + 1], v1i[1], v1i[slot + 1])
        bv, bi = vals[0], idxs[0]
        for qq in range(1, NQ):
          ov, oi = vals[qq], idxs[qq]
          take = (ov > bv) | ((ov == bv) & (oi < bi))
          bv = jnp.where(take, ov, bv)
          bi = jnp.where(take, oi, bi)
        return (bi, acc)

      far, acc = lax.fori_loop(
          0, L, fps_iter, (far0, jnp.zeros((L,), jnp.int32)))
      idxb[pl.ds(blk * L, L)] = acc
      return far

    lax.fori_loop(0, npoint // L, fps_block, jnp.int32(0))

    obase = q * opq

    def gather_step(j, carry):
      sl = pl.ds(j * L, L)
      iv = idxb[pl.ds(obase + j * L, L)]
      oxs[sl] = plsc.load_gather(xs, [iv])
      oys[sl] = plsc.load_gather(ys, [iv])
      ozs[sl] = plsc.load_gather(zs, [iv])
      oms[sl] = plsc.load_gather(msk, [iv])
      idx2[sl] = iv + b * n
      return carry

    lax.fori_loop(0, opq // L, gather_step, 0)

    pltpu.sync_copy(oxs, oc_hbm.at[pl.ds((b * 3 + 0) * npoint + obase, opq)])
    pltpu.sync_copy(oys, oc_hbm.at[pl.ds((b * 3 + 1) * npoint + obase, opq)])
    pltpu.sync_copy(ozs, oc_hbm.at[pl.ds((b * 3 + 2) * npoint + obase, opq)])
    pltpu.sync_copy(oms, om_hbm.at[pl.ds(b * npoint + obase, opq)])

    pltpu.async_copy(vt_hbm.at[idx2], vbuf, sem).wait()
    pltpu.sync_copy(vbuf, ov_hbm.at[b, pl.ds(obase, opq)])

  return _fps_body


def _make_call(b_sz, n, npoint, dv, interpret=False):
  mesh = plsc.VectorSubcoreMesh(
      core_axis_name="c", subcore_axis_name="s", num_cores=2, num_subcores=16)
  nq_pts = n // NQ
  opq = npoint // NQ
  slot = 8
  return pl.kernel(
      _make_body(b_sz, n, npoint, dv),
      out_type=[
          jax.ShapeDtypeStruct((b_sz * 3 * npoint,), jnp.float32),
          jax.ShapeDtypeStruct((b_sz, npoint, dv), jnp.float32),
          jax.ShapeDtypeStruct((b_sz * npoint,), jnp.float32),
      ],
      mesh=mesh,
      scratch_types=[
          pltpu.VMEM((n,), jnp.float32),
          pltpu.VMEM((n,), jnp.float32),
          pltpu.VMEM((n,), jnp.float32),
          pltpu.VMEM((n,), jnp.float32),
          pltpu.VMEM((n,), jnp.float32),
          pltpu.VMEM((n,), jnp.float32),
          pltpu.VMEM((n,), jnp.float32),
          pltpu.VMEM((nq_pts,), jnp.float32),
          pltpu.VMEM((n,), jnp.float32),
          pltpu.VMEM((npoint,), jnp.int32),
          pltpu.VMEM((opq,), jnp.int32),
          pltpu.VMEM((opq,), jnp.float32),
          pltpu.VMEM((opq,), jnp.float32),
          pltpu.VMEM((opq,), jnp.float32),
          pltpu.VMEM((opq,), jnp.float32),
          pltpu.VMEM((L,), jnp.int32),
          pltpu.VMEM((NQ * slot,), jnp.int32),
          pltpu.VMEM((opq, dv), jnp.float32),
          pltpu.VMEM_SHARED((2 * 16 * slot,), jnp.int32),
          pltpu.SemaphoreType.DMA,
      ],
      compiler_params=pltpu.CompilerParams(
          needs_layout_passes=False, use_tc_tiling_on_sc=False),
      interpret=interpret,
  )


@jax.jit
def _fps_call(ct, vt, mt):
  return _make_call(B, N, NP, DV)(ct, vt, mt)


def kernel(coords, values, mask):
  ct = coords.transpose(0, 2, 1).reshape(B * 3 * N)
  vt = values.reshape(B * N, DV)
  mt = mask.reshape(B * N)
  oc, ov, om = _fps_call(ct, vt, mt)
  return (oc.reshape(B, 3, NP).transpose(0, 2, 1), ov,
          om.reshape(B, NP, 1))
